# baseline (speedup 1.0000x reference)
import functools

import jax
import jax.numpy as jnp
from jax.experimental import pallas as pl
from jax.experimental.pallas import tpu as pltpu

N_NEI = 4
D = 2048
NV = 20
B = 32
M = B * NV
BN = 512


def _fused_body(x_ref, w_ref, b_ref, out_ref, xbf_ref, adj_ref,
                s0_ref, s1_ref, s2_ref, s3_ref):
    j = pl.program_id(0)
    s_refs = (s0_ref, s1_ref, s2_ref, s3_ref)

    @pl.when(j == 0)
    def _select():
        xv = x_ref[...]
        xbf_ref[...] = xv.reshape(M, D).astype(jnp.bfloat16)
        for b in range(B):
            xb = xv[b]
            g = jax.lax.dot_general(xb, xb, (((1,), (1,)), ((), ())),
                                    preferred_element_type=jnp.float32)
            sq = jnp.sum(xb * xb, axis=1)
            inner = -2.0 * g
            adj_ref[b * NV:(b + 1) * NV, :] = \
                (sq[None, :] + inner) + sq[:, None]

        cur = adj_ref[...]
        lane = jax.lax.broadcasted_iota(jnp.int32, (M, NV), 1)
        for k in range(N_NEI):
            mn = jnp.min(cur, axis=1, keepdims=True)
            eq = cur == mn
            idx = jnp.min(jnp.where(eq, lane, NV), axis=1, keepdims=True)
            oh = lane == idx
            s_refs[k][...] = oh.astype(jnp.float32)
            if k + 1 < N_NEI:
                cur = jnp.where(oh, jnp.float32(jnp.inf), cur)

    xbf = xbf_ref[...]
    wbf = w_ref[...].astype(jnp.bfloat16)
    dn = (((1,), (1,)), ((), ()))
    c1 = jax.lax.dot_general(xbf, wbf[:, :D], dn,
                             preferred_element_type=jnp.float32)
    c2 = jax.lax.dot_general(xbf, wbf[:, D:], dn,
                             preferred_element_type=jnp.float32)
    d = c1 - c2 + b_ref[...]

    for b in range(B):
        rows = slice(b * NV, (b + 1) * NV)
        c2b = c2[rows]
        p = None
        for k in range(N_NEI):
            sk = s_refs[k][rows, :]
            pk = jax.lax.dot_general(sk, c2b, (((1,), (0,)), ((), ())),
                                     preferred_element_type=jnp.float32)
            p = pk if p is None else jnp.maximum(p, pk)
        h = jnp.maximum(d[rows] + p, 0.0)
        out_ref[b, :] = jnp.max(h, axis=0)


@functools.partial(jax.jit, static_argnames=())
def kernel(x, W, b):
    b2d = b.reshape(1, D)

    out = pl.pallas_call(
        _fused_body,
        grid=(D // BN,),
        in_specs=[
            pl.BlockSpec((B, NV, D), lambda j: (0, 0, 0)),
            pl.BlockSpec((BN, 2 * D), lambda j: (j, 0)),
            pl.BlockSpec((1, BN), lambda j: (0, j)),
        ],
        out_specs=pl.BlockSpec((B, BN), lambda j: (0, j)),
        out_shape=jax.ShapeDtypeStruct((B, D), jnp.float32),
        scratch_shapes=[
            pltpu.VMEM((M, D), jnp.bfloat16),
            pltpu.VMEM((M, NV), jnp.float32),
            pltpu.VMEM((M, NV), jnp.float32),
            pltpu.VMEM((M, NV), jnp.float32),
            pltpu.VMEM((M, NV), jnp.float32),
            pltpu.VMEM((M, NV), jnp.float32),
        ],
    )(x, W, b2d)

    return out.reshape(B, D, 1, 1)
